# baseline (speedup 1.0000x reference)
import functools

import jax
import jax.numpy as jnp
from jax import lax
from jax.experimental import pallas as pl
from jax.experimental.pallas import tpu as pltpu
from jax.experimental.pallas import tpu_sc as plsc

NC, NS = 2, 16
NW = NC * NS
ROWS = 4096 * 200
D = 64
BPW = ROWS // NW
GROUP = 512
CHUNK = 128
NCHUNK = GROUP // CHUNK
NGROUPS = BPW // GROUP

_mesh = plsc.VectorSubcoreMesh(core_axis_name="c", subcore_axis_name="s")


@functools.partial(
    pl.kernel,
    out_type=jax.ShapeDtypeStruct((ROWS, D), jnp.float32),
    mesh=_mesh,
    compiler_params=pltpu.CompilerParams(use_tc_tiling_on_sc=False),
    scratch_types=[
        pltpu.VMEM((GROUP,), jnp.int32),
        pltpu.VMEM((GROUP,), jnp.int32),
        pltpu.VMEM((GROUP, D), jnp.float32),
        pltpu.VMEM((GROUP, D), jnp.float32),
        pltpu.SemaphoreType.DMA,
        pltpu.SemaphoreType.DMA,
    ],
)
def _emb_lookup(x_hbm, table_hbm, out_hbm, idx0, idx1, rows0, rows1, sem0, sem1):
    wid = lax.axis_index("s") * NC + lax.axis_index("c")
    base = wid * BPW
    idx = (idx0, idx1)
    rows = (rows0, rows1)
    sem = (sem0, sem1)

    def fire(g, b):
        pltpu.sync_copy(x_hbm.at[pl.ds(base + g * GROUP, GROUP)], idx[b])
        for j in range(NCHUNK):
            pltpu.async_copy(
                table_hbm.at[idx[b].at[pl.ds(j * CHUNK, CHUNK)]],
                rows[b].at[pl.ds(j * CHUNK, CHUNK)],
                sem[b],
            )

    def drain_write(g, b):
        pltpu.make_async_copy(table_hbm.at[pl.ds(0, GROUP)], rows[b], sem[b]).wait()
        pltpu.sync_copy(rows[b], out_hbm.at[pl.ds(base + g * GROUP, GROUP)])

    fire(0, 0)

    def body(t, carry):
        for b in range(2):
            g = 2 * t + b

            @pl.when(g + 1 < NGROUPS)
            def _():
                fire(g + 1, 1 - b)

            drain_write(g, b)
        return carry

    lax.fori_loop(0, NGROUPS // 2, body, 0)


def kernel(x, table):
    out = _emb_lookup(jnp.reshape(x, (ROWS,)), table)
    return jnp.reshape(out, (4096, 200, D))
